# baseline (speedup 1.0000x reference)
import functools

import jax
import jax.numpy as jnp
from jax import lax
from jax.experimental import pallas as pl
from jax.experimental.pallas import tpu as pltpu
from jax.experimental.pallas import tpu_sc as plsc

D_MODEL = 1024
N_HEADS = 16
D_HEAD = 64
SEQ = 4096
BUCKET = 64
N_HASHES = 4
RADIUS = 4
NB_HALF = 32
N_CODES = 2 * NB_HALF
SCALE = 1.0 / (D_HEAD ** 0.5)
ROW_TILE = 512
N_ROW_TILES = SEQ // ROW_TILE
CSUM_CHUNK = 128
N_CSUM_CHUNKS = SEQ // CSUM_CHUNK
ATT_GROUP = 4 * BUCKET
N_COMBOS = N_HEADS * N_HASHES
NC, NS = 2, 16
N_WORKERS = NC * NS
GCHUNK = 1024


def _layernorm(x, g, b):
    mu = jnp.mean(x, axis=-1, keepdims=True)
    var = jnp.mean((x - mu) ** 2, axis=-1, keepdims=True)
    return (x - mu) / jnp.sqrt(var + 1e-5) * g + b


def _proj_body(x2_ref, g_ref, b_ref, wqk_ref, wv_ref, qk3_ref, v3_ref):
    x = x2_ref[...]
    h = _layernorm(x, g_ref[...], b_ref[...])
    qk = jnp.dot(h, wqk_ref[...], preferred_element_type=jnp.float32)
    v = jnp.dot(h.astype(jnp.bfloat16), wv_ref[...],
                preferred_element_type=jnp.float32)
    for hh in range(N_HEADS):
        qk3_ref[hh] = qk[:, hh * D_HEAD:(hh + 1) * D_HEAD]
        v3_ref[hh] = v[:, hh * D_HEAD:(hh + 1) * D_HEAD]


def _stage1(x2d, g, b, Wqk, Wv):
    return pl.pallas_call(
        _proj_body,
        grid=(N_ROW_TILES,),
        in_specs=[
            pl.BlockSpec((ROW_TILE, D_MODEL), lambda i: (i, 0)),
            pl.BlockSpec((1, D_MODEL), lambda i: (0, 0)),
            pl.BlockSpec((1, D_MODEL), lambda i: (0, 0)),
            pl.BlockSpec((D_MODEL, D_MODEL), lambda i: (0, 0)),
            pl.BlockSpec((D_MODEL, D_MODEL), lambda i: (0, 0)),
        ],
        out_specs=[
            pl.BlockSpec((N_HEADS, ROW_TILE, D_HEAD), lambda i: (0, i, 0)),
            pl.BlockSpec((N_HEADS, ROW_TILE, D_HEAD), lambda i: (0, i, 0)),
        ],
        out_shape=[
            jax.ShapeDtypeStruct((N_HEADS, SEQ, D_HEAD), jnp.float32),
            jax.ShapeDtypeStruct((N_HEADS, SEQ, D_HEAD), jnp.float32),
        ],
    )(x2d, g, b, Wqk, Wv.astype(jnp.bfloat16))


def _codes_body(qk3_ref, rhT_ref, inv_ref, scat_ref):
    h = pl.program_id(0)
    qk = qk3_ref[0]
    rotT_all = lax.dot_general(rhT_ref[...], qk, (((1,), (1,)), ((), ())),
                               preferred_element_type=jnp.float32)
    iota_sub32 = lax.broadcasted_iota(jnp.int32, (NB_HALF, SEQ), 0)
    iota_sub64 = lax.broadcasted_iota(jnp.int32, (N_CODES, SEQ), 0)
    utri128 = (lax.broadcasted_iota(jnp.int32, (CSUM_CHUNK, CSUM_CHUNK), 0)
               <= lax.broadcasted_iota(jnp.int32, (CSUM_CHUNK, CSUM_CHUNK), 1)
               ).astype(jnp.bfloat16)
    ustrict32 = (lax.broadcasted_iota(jnp.int32, (N_CSUM_CHUNKS, N_CSUM_CHUNKS), 0)
                 < lax.broadcasted_iota(jnp.int32, (N_CSUM_CHUNKS, N_CSUM_CHUNKS), 1)
                 ).astype(jnp.bfloat16)
    lstrict64 = (lax.broadcasted_iota(jnp.int32, (N_CODES, N_CODES), 0)
                 > lax.broadcasted_iota(jnp.int32, (N_CODES, N_CODES), 1)
                 ).astype(jnp.float32)
    for r in range(N_HASHES):
        rotT = rotT_all[r * NB_HALF:(r + 1) * NB_HALF]
        m1 = jnp.max(rotT, axis=0, keepdims=True)
        m2 = jnp.min(rotT, axis=0, keepdims=True)
        am1 = jnp.min(jnp.where(rotT == m1, iota_sub32, NB_HALF), axis=0,
                      keepdims=True)
        am2 = jnp.min(jnp.where(rotT == m2, iota_sub32, NB_HALF), axis=0,
                      keepdims=True)
        code = jnp.where(m1 >= -m2, am1, NB_HALF + am2)
        ohb = (iota_sub64 == code).astype(jnp.bfloat16)
        oh = ohb.astype(jnp.float32)
        chunks = []
        sums = []
        for c in range(N_CSUM_CHUNKS):
            oc = ohb[:, c * CSUM_CHUNK:(c + 1) * CSUM_CHUNK]
            wc = jnp.dot(oc, utri128, preferred_element_type=jnp.float32)
            chunks.append(wc)
            sums.append(wc[:, CSUM_CHUNK - 1:CSUM_CHUNK])
        sums = jnp.concatenate(sums, axis=1)
        offs = jnp.dot(sums.astype(jnp.bfloat16), ustrict32,
                       preferred_element_type=jnp.float32)
        csum = jnp.concatenate(
            [chunks[c] + offs[:, c:c + 1] for c in range(N_CSUM_CHUNKS)],
            axis=1)
        tot = jnp.sum(sums, axis=1, keepdims=True)
        code_off = jnp.dot(lstrict64, tot, preferred_element_type=jnp.float32)
        inv_f = jnp.sum(oh * (csum + code_off), axis=0, keepdims=True) - 1.0
        inv_i = inv_f.astype(jnp.int32)[0]
        inv_ref[0, r] = inv_i
        scat_ref[0, r] = inv_i + (h * N_HASHES + r) * SEQ


def _stage2(qk3, RhT):
    return pl.pallas_call(
        _codes_body,
        grid=(N_HEADS,),
        in_specs=[
            pl.BlockSpec((1, SEQ, D_HEAD), lambda h: (h, 0, 0)),
            pl.BlockSpec((N_HASHES * NB_HALF, D_HEAD), lambda h: (0, 0)),
        ],
        out_specs=[
            pl.BlockSpec((1, N_HASHES, SEQ), lambda h: (h, 0, 0)),
            pl.BlockSpec((1, N_HASHES, SEQ), lambda h: (h, 0, 0)),
        ],
        out_shape=[
            jax.ShapeDtypeStruct((N_HEADS, N_HASHES, SEQ), jnp.int32),
            jax.ShapeDtypeStruct((N_HEADS, N_HASHES, SEQ), jnp.int32),
        ],
    )(qk3, RhT)


def _sc_route_body(qk_hbm, v_hbm, inv_hbm, sqk_hbm, sv_hbm,
                   inv_v, perm_v, buf_v, sem):
    wid = lax.axis_index("s") * NC + lax.axis_index("c")
    iota16 = jnp.arange(16, dtype=jnp.int32)
    for t in range(N_COMBOS // N_WORKERS):
        combo = wid * (N_COMBOS // N_WORKERS) + t
        hh = combo // N_HASHES
        rr = combo % N_HASHES
        pltpu.sync_copy(inv_hbm.at[hh, rr], inv_v)
        base = hh * SEQ

        def body(i, carry):
            idx = inv_v[pl.ds(i * 16, 16)]
            vals = base + i * 16 + iota16
            plsc.store_scatter(perm_v, [idx], vals)
            return carry

        lax.fori_loop(0, SEQ // 16, body, 0)
        for cc in range(SEQ // GCHUNK):
            pidx = perm_v.at[pl.ds(cc * GCHUNK, GCHUNK)]
            pltpu.async_copy(qk_hbm.at[pidx], buf_v, sem).wait()
            pltpu.sync_copy(
                buf_v, sqk_hbm.at[hh, rr, pl.ds(cc * GCHUNK, GCHUNK)])
            pltpu.async_copy(v_hbm.at[pidx], buf_v, sem).wait()
            pltpu.sync_copy(
                buf_v, sv_hbm.at[hh, rr, pl.ds(cc * GCHUNK, GCHUNK)])


def _sc_route(qk_flat, v_flat, inv4):
    mesh = plsc.VectorSubcoreMesh(core_axis_name="c", subcore_axis_name="s")
    f = functools.partial(
        pl.kernel,
        mesh=mesh,
        compiler_params=pltpu.CompilerParams(needs_layout_passes=False,
                                             use_tc_tiling_on_sc=False),
        out_type=[
            jax.ShapeDtypeStruct((N_HEADS, N_HASHES, SEQ, D_HEAD), jnp.bfloat16),
            jax.ShapeDtypeStruct((N_HEADS, N_HASHES, SEQ, D_HEAD), jnp.bfloat16),
        ],
        scratch_types=[
            pltpu.VMEM((SEQ,), jnp.int32),
            pltpu.VMEM((SEQ,), jnp.int32),
            pltpu.VMEM((GCHUNK, D_HEAD), jnp.bfloat16),
            pltpu.SemaphoreType.DMA,
        ],
    )(_sc_route_body)
    return f(qk_flat, v_flat, inv4)


def _bucket_body(sqk_ref, sv_ref, ob_ref):
    iota_r = lax.broadcasted_iota(jnp.int32, (ATT_GROUP, ATT_GROUP), 0)
    iota_cc = lax.broadcasted_iota(jnp.int32, (ATT_GROUP, ATT_GROUP), 1)
    same_chunk = (iota_r // BUCKET) == (iota_cc // BUCKET)
    for g in range(SEQ // ATT_GROUP):
        sq = sqk_ref[0, 0, g * ATT_GROUP:(g + 1) * ATT_GROUP].astype(
            jnp.bfloat16)
        sv = sv_ref[0, 0, g * ATT_GROUP:(g + 1) * ATT_GROUP].astype(
            jnp.bfloat16)
        s = lax.dot_general(sq, sq, (((1,), (1,)), ((), ())),
                            preferred_element_type=jnp.float32)
        s = jnp.where(same_chunk, s * SCALE, -1e9)
        m = jnp.max(s, axis=1, keepdims=True)
        e = jnp.exp(s - m)
        p = e / jnp.sum(e, axis=1, keepdims=True)
        ob_ref[0, 0, g * ATT_GROUP:(g + 1) * ATT_GROUP] = jnp.dot(
            p.astype(jnp.bfloat16), sv, preferred_element_type=jnp.float32)


def _stage3(sqk, sv):
    return pl.pallas_call(
        _bucket_body,
        grid=(N_HEADS, N_HASHES),
        in_specs=[
            pl.BlockSpec((1, 1, SEQ, D_HEAD), lambda h, r: (h, r, 0, 0)),
            pl.BlockSpec((1, 1, SEQ, D_HEAD), lambda h, r: (h, r, 0, 0)),
        ],
        out_specs=pl.BlockSpec((1, 1, SEQ, D_HEAD), lambda h, r: (h, r, 0, 0)),
        out_shape=jax.ShapeDtypeStruct((N_HEADS, N_HASHES, SEQ, D_HEAD),
                                       jnp.float32),
    )(sqk, sv)


def _sc_unsort_body(ob_hbm, scat_hbm, acc_hbm, idx_v, buf_v, sem):
    wid = lax.axis_index("s") * NC + lax.axis_index("c")
    for t in range(N_COMBOS // N_WORKERS):
        combo = wid * (N_COMBOS // N_WORKERS) + t
        hh = combo // N_HASHES
        rr = combo % N_HASHES
        pltpu.sync_copy(scat_hbm.at[hh, rr], idx_v)
        for cc in range(SEQ // GCHUNK):
            sidx = idx_v.at[pl.ds(cc * GCHUNK, GCHUNK)]
            pltpu.async_copy(ob_hbm.at[sidx], buf_v, sem).wait()
            pltpu.sync_copy(
                buf_v, acc_hbm.at[hh, rr, pl.ds(cc * GCHUNK, GCHUNK)])


def _sc_unsort(ob_flat, scat4):
    mesh = plsc.VectorSubcoreMesh(core_axis_name="c", subcore_axis_name="s")
    f = functools.partial(
        pl.kernel,
        mesh=mesh,
        compiler_params=pltpu.CompilerParams(needs_layout_passes=False,
                                             use_tc_tiling_on_sc=False),
        out_type=jax.ShapeDtypeStruct((N_HEADS, N_HASHES, SEQ, D_HEAD),
                                      jnp.float32),
        scratch_types=[
            pltpu.VMEM((SEQ,), jnp.int32),
            pltpu.VMEM((GCHUNK, D_HEAD), jnp.float32),
            pltpu.SemaphoreType.DMA,
        ],
    )(_sc_unsort_body)
    return f(ob_flat, scat4)


LCHUNK = 128
HALO = 8
HW = LCHUNK + 2 * HALO


def _mix_body(qk3_ref, v3_ref, acc_ref, wr_ref, mixed_ref, reg_ref):
    qk = qk3_ref[0]
    v = v3_ref[0]
    zp = jnp.zeros((HALO, D_HEAD), jnp.float32)
    qk_pad = jnp.concatenate([zp, qk, zp], axis=0).astype(jnp.bfloat16)
    v_pad = jnp.concatenate([zp, v, zp], axis=0).astype(jnp.bfloat16)
    qkb = qk.astype(jnp.bfloat16)
    di = (lax.broadcasted_iota(jnp.int32, (LCHUNK, HW), 1)
          - lax.broadcasted_iota(jnp.int32, (LCHUNK, HW), 0) - HALO)
    band = (di >= -RADIUS) & (di <= RADIUS)
    gcol_j = lax.broadcasted_iota(jnp.int32, (LCHUNK, HW), 1)
    locs = []
    for c in range(SEQ // LCHUNK):
        kh = qk_pad[c * LCHUNK:c * LCHUNK + HW]
        vh = v_pad[c * LCHUNK:c * LCHUNK + HW]
        s = lax.dot_general(qkb[c * LCHUNK:(c + 1) * LCHUNK], kh,
                            (((1,), (1,)), ((), ())),
                            preferred_element_type=jnp.float32)
        mask = band
        if c == 0 or c == SEQ // LCHUNK - 1:
            gcol = c * LCHUNK - HALO + gcol_j
            mask = mask & (gcol >= 0) & (gcol < SEQ)
        s = jnp.where(mask, s * SCALE, -1e9)
        m = jnp.max(s, axis=1, keepdims=True)
        e = jnp.exp(s - m)
        p = e / jnp.sum(e, axis=1, keepdims=True)
        locs.append(jnp.dot(p.astype(jnp.bfloat16), vh,
                            preferred_element_type=jnp.float32))
    local = jnp.concatenate(locs, axis=0)
    glin = jnp.dot(qk, wr_ref[0], preferred_element_type=jnp.float32)
    gm = jnp.max(glin, axis=1, keepdims=True)
    ge = jnp.exp(glin - gm)
    gate = ge / jnp.sum(ge, axis=1, keepdims=True)
    reg_ref[0] = jnp.sum(gate * jnp.log(gate + 1e-9), keepdims=True)
    lsh = (acc_ref[0, 0] + acc_ref[0, 1] + acc_ref[0, 2] + acc_ref[0, 3]) * 0.25
    mixed_ref[0] = (gate[:, 0:1] * local + gate[:, 1:2] * lsh).astype(
        jnp.bfloat16)


def _stage4(qk3, v3, acc, Wr):
    return pl.pallas_call(
        _mix_body,
        grid=(N_HEADS,),
        in_specs=[
            pl.BlockSpec((1, SEQ, D_HEAD), lambda h: (h, 0, 0)),
            pl.BlockSpec((1, SEQ, D_HEAD), lambda h: (h, 0, 0)),
            pl.BlockSpec((1, N_HASHES, SEQ, D_HEAD), lambda h: (h, 0, 0, 0)),
            pl.BlockSpec((1, D_HEAD, 2), lambda h: (h, 0, 0)),
        ],
        out_specs=[
            pl.BlockSpec((1, SEQ, D_HEAD), lambda h: (h, 0, 0)),
            pl.BlockSpec((1, 1, 1), lambda h: (h, 0, 0)),
        ],
        out_shape=[
            jax.ShapeDtypeStruct((N_HEADS, SEQ, D_HEAD), jnp.bfloat16),
            jax.ShapeDtypeStruct((N_HEADS, 1, 1), jnp.float32),
        ],
    )(qk3, v3, acc, Wr)


def _out_body(mixed_ref, x1_ref, x2_ref, wo_ref, w1_ref, b1_ref, w2_ref,
              b2_ref, g_ref, b_ref, gf_ref, gg_ref, y1_ref, y2_ref):
    mixed = jnp.concatenate([mixed_ref[hh] for hh in range(N_HEADS)], axis=1)
    attn = jnp.dot(mixed, wo_ref[...], preferred_element_type=jnp.float32)
    sig_f = 1.0 / (1.0 + jnp.exp(-gf_ref[...]))
    sig_g = 1.0 / (1.0 + jnp.exp(-gg_ref[...]))
    y1 = x1_ref[...] + sig_f * attn
    hn = _layernorm(y1, g_ref[...], b_ref[...])
    f = jnp.maximum(
        jnp.dot(hn.astype(jnp.bfloat16), w1_ref[...],
                preferred_element_type=jnp.float32) + b1_ref[...], 0.0)
    ffn = jnp.dot(f.astype(jnp.bfloat16), w2_ref[...],
                  preferred_element_type=jnp.float32) + b2_ref[...]
    y1_ref[...] = y1
    y2_ref[...] = x2_ref[...] + sig_g * ffn


def _stage5(mixed3, x1d, x2d, Wo, W1, b1, W2, b2, g, b, gf, gg):
    Wo = Wo.astype(jnp.bfloat16)
    W1 = W1.astype(jnp.bfloat16)
    W2 = W2.astype(jnp.bfloat16)
    vec = pl.BlockSpec((1, D_MODEL), lambda i: (0, 0))
    mat = pl.BlockSpec((D_MODEL, D_MODEL), lambda i: (0, 0))
    row = pl.BlockSpec((ROW_TILE, D_MODEL), lambda i: (i, 0))
    return pl.pallas_call(
        _out_body,
        grid=(N_ROW_TILES,),
        in_specs=[
            pl.BlockSpec((N_HEADS, ROW_TILE, D_HEAD), lambda i: (0, i, 0)),
            row, row, mat, mat, vec, mat, vec, vec, vec, vec, vec,
        ],
        out_specs=[row, row],
        out_shape=[
            jax.ShapeDtypeStruct((SEQ, D_MODEL), jnp.float32),
            jax.ShapeDtypeStruct((SEQ, D_MODEL), jnp.float32),
        ],
    )(mixed3, x1d, x2d, Wo, W1, b1, W2, b2, g, b, gf, gg)


def kernel(x1, x2, Wqk, Wv, Wo, Rh, Wr, ln_attn_g, ln_attn_b, ln_ffn_g,
           ln_ffn_b, W1, b1, W2, b2, gate_f, gate_g):
    x1d = x1[0]
    x2d = x2[0]
    qk3, v3 = _stage1(x2d, ln_attn_g[None], ln_attn_b[None], Wqk, Wv)
    RhT = Rh.transpose(0, 2, 1).reshape(N_HASHES * NB_HALF, D_HEAD)
    inv4, scat4 = _stage2(qk3, RhT)
    qk_flat = qk3.reshape(N_HEADS * SEQ, D_HEAD).astype(jnp.bfloat16)
    v_flat = v3.reshape(N_HEADS * SEQ, D_HEAD).astype(jnp.bfloat16)
    sqk, sv = _sc_route(qk_flat, v_flat, inv4)
    ob = _stage3(sqk, sv)
    ob_flat = ob.reshape(N_COMBOS * SEQ, D_HEAD)
    acc = _sc_unsort(ob_flat, scat4)
    mixed3, regs = _stage4(qk3, v3, acc, Wr)
    y1d, y2d = _stage5(mixed3, x1d, x2d, Wo, W1, b1[None], W2, b2[None],
                       ln_ffn_g[None], ln_ffn_b[None], gate_f[None],
                       gate_g[None])
    reg_loss = -jnp.sum(regs) / (N_HEADS * SEQ)
    return (y1d[None], y2d[None], reg_loss)


# speedup vs baseline: 1.1144x; 1.1144x over previous
import functools

import jax
import jax.numpy as jnp
from jax import lax
from jax.experimental import pallas as pl
from jax.experimental.pallas import tpu as pltpu
from jax.experimental.pallas import tpu_sc as plsc

D_MODEL = 1024
N_HEADS = 16
D_HEAD = 64
SEQ = 4096
BUCKET = 64
N_HASHES = 4
RADIUS = 4
NB_HALF = 32
N_CODES = 2 * NB_HALF
SCALE = 1.0 / (D_HEAD ** 0.5)
ROW_TILE = 512
N_ROW_TILES = SEQ // ROW_TILE
CSUM_CHUNK = 128
N_CSUM_CHUNKS = SEQ // CSUM_CHUNK
ATT_GROUP = 4 * BUCKET
N_COMBOS = N_HEADS * N_HASHES
NC, NS = 2, 16
N_WORKERS = NC * NS
GCHUNK = 1024


def _layernorm(x, g, b):
    mu = jnp.mean(x, axis=-1, keepdims=True)
    var = jnp.mean((x - mu) ** 2, axis=-1, keepdims=True)
    return (x - mu) / jnp.sqrt(var + 1e-5) * g + b


def _proj_body(x2_ref, g_ref, b_ref, wqk_ref, wv_ref, qk3_ref, v3_ref):
    x = x2_ref[...]
    h = _layernorm(x, g_ref[...], b_ref[...])
    qk = jnp.dot(h, wqk_ref[...], preferred_element_type=jnp.float32)
    v = jnp.dot(h.astype(jnp.bfloat16), wv_ref[...],
                preferred_element_type=jnp.float32)
    for hh in range(N_HEADS):
        qk3_ref[hh] = qk[:, hh * D_HEAD:(hh + 1) * D_HEAD]
        v3_ref[hh] = v[:, hh * D_HEAD:(hh + 1) * D_HEAD]


def _stage1(x2d, g, b, Wqk, Wv):
    return pl.pallas_call(
        _proj_body,
        grid=(N_ROW_TILES,),
        in_specs=[
            pl.BlockSpec((ROW_TILE, D_MODEL), lambda i: (i, 0)),
            pl.BlockSpec((1, D_MODEL), lambda i: (0, 0)),
            pl.BlockSpec((1, D_MODEL), lambda i: (0, 0)),
            pl.BlockSpec((D_MODEL, D_MODEL), lambda i: (0, 0)),
            pl.BlockSpec((D_MODEL, D_MODEL), lambda i: (0, 0)),
        ],
        out_specs=[
            pl.BlockSpec((N_HEADS, ROW_TILE, D_HEAD), lambda i: (0, i, 0)),
            pl.BlockSpec((N_HEADS, ROW_TILE, D_HEAD), lambda i: (0, i, 0)),
        ],
        out_shape=[
            jax.ShapeDtypeStruct((N_HEADS, SEQ, D_HEAD), jnp.float32),
            jax.ShapeDtypeStruct((N_HEADS, SEQ, D_HEAD), jnp.float32),
        ],
    )(x2d, g, b, Wqk, Wv.astype(jnp.bfloat16))


def _codes_body(qk3_ref, rhT_ref, inv_ref, scat_ref):
    h = pl.program_id(0)
    qk = qk3_ref[0]
    rotT_all = lax.dot_general(rhT_ref[...], qk, (((1,), (1,)), ((), ())),
                               preferred_element_type=jnp.float32)
    iota_sub32 = lax.broadcasted_iota(jnp.int32, (NB_HALF, SEQ), 0)
    iota_sub64 = lax.broadcasted_iota(jnp.int32, (N_CODES, SEQ), 0)
    utri128 = (lax.broadcasted_iota(jnp.int32, (CSUM_CHUNK, CSUM_CHUNK), 0)
               <= lax.broadcasted_iota(jnp.int32, (CSUM_CHUNK, CSUM_CHUNK), 1)
               ).astype(jnp.bfloat16)
    ustrict32 = (lax.broadcasted_iota(jnp.int32, (N_CSUM_CHUNKS, N_CSUM_CHUNKS), 0)
                 < lax.broadcasted_iota(jnp.int32, (N_CSUM_CHUNKS, N_CSUM_CHUNKS), 1)
                 ).astype(jnp.bfloat16)
    lstrict64 = (lax.broadcasted_iota(jnp.int32, (N_CODES, N_CODES), 0)
                 > lax.broadcasted_iota(jnp.int32, (N_CODES, N_CODES), 1)
                 ).astype(jnp.float32)
    for r in range(N_HASHES):
        rotT = rotT_all[r * NB_HALF:(r + 1) * NB_HALF]
        m1 = jnp.max(rotT, axis=0, keepdims=True)
        m2 = jnp.min(rotT, axis=0, keepdims=True)
        am1 = jnp.min(jnp.where(rotT == m1, iota_sub32, NB_HALF), axis=0,
                      keepdims=True)
        am2 = jnp.min(jnp.where(rotT == m2, iota_sub32, NB_HALF), axis=0,
                      keepdims=True)
        code = jnp.where(m1 >= -m2, am1, NB_HALF + am2)
        ohb = (iota_sub64 == code).astype(jnp.bfloat16)
        oh = ohb.astype(jnp.float32)
        chunks = []
        sums = []
        for c in range(N_CSUM_CHUNKS):
            oc = ohb[:, c * CSUM_CHUNK:(c + 1) * CSUM_CHUNK]
            wc = jnp.dot(oc, utri128, preferred_element_type=jnp.float32)
            chunks.append(wc)
            sums.append(wc[:, CSUM_CHUNK - 1:CSUM_CHUNK])
        sums = jnp.concatenate(sums, axis=1)
        offs = jnp.dot(sums.astype(jnp.bfloat16), ustrict32,
                       preferred_element_type=jnp.float32)
        csum = jnp.concatenate(
            [chunks[c] + offs[:, c:c + 1] for c in range(N_CSUM_CHUNKS)],
            axis=1)
        tot = jnp.sum(sums, axis=1, keepdims=True)
        code_off = jnp.dot(lstrict64, tot, preferred_element_type=jnp.float32)
        inv_f = jnp.sum(oh * (csum + code_off), axis=0, keepdims=True) - 1.0
        inv_i = inv_f.astype(jnp.int32)[0]
        inv_ref[0, r] = inv_i
        scat_ref[0, r] = inv_i + (h * N_HASHES + r) * SEQ


def _stage2(qk3, RhT):
    return pl.pallas_call(
        _codes_body,
        grid=(N_HEADS,),
        in_specs=[
            pl.BlockSpec((1, SEQ, D_HEAD), lambda h: (h, 0, 0)),
            pl.BlockSpec((N_HASHES * NB_HALF, D_HEAD), lambda h: (0, 0)),
        ],
        out_specs=[
            pl.BlockSpec((1, N_HASHES, SEQ), lambda h: (h, 0, 0)),
            pl.BlockSpec((1, N_HASHES, SEQ), lambda h: (h, 0, 0)),
        ],
        out_shape=[
            jax.ShapeDtypeStruct((N_HEADS, N_HASHES, SEQ), jnp.int32),
            jax.ShapeDtypeStruct((N_HEADS, N_HASHES, SEQ), jnp.int32),
        ],
    )(qk3, RhT)


def _sc_route_body(qk_hbm, v_hbm, inv_hbm, sqk_hbm, sv_hbm,
                   inv_v, perm_v, buf_v, sem):
    wid = lax.axis_index("s") * NC + lax.axis_index("c")
    iota16 = jnp.arange(16, dtype=jnp.int32)
    for t in range(N_COMBOS // N_WORKERS):
        combo = wid * (N_COMBOS // N_WORKERS) + t
        hh = combo // N_HASHES
        rr = combo % N_HASHES
        pltpu.sync_copy(inv_hbm.at[hh, rr], inv_v)
        base = hh * SEQ

        def body(i, carry):
            idx = inv_v[pl.ds(i * 16, 16)]
            vals = base + i * 16 + iota16
            plsc.store_scatter(perm_v, [idx], vals)
            return carry

        lax.fori_loop(0, SEQ // 16, body, 0)
        for cc in range(SEQ // GCHUNK):
            pidx = perm_v.at[pl.ds(cc * GCHUNK, GCHUNK)]
            pltpu.async_copy(qk_hbm.at[pidx], buf_v, sem).wait()
            pltpu.sync_copy(
                buf_v, sqk_hbm.at[hh, rr, pl.ds(cc * GCHUNK, GCHUNK)])
            pltpu.async_copy(v_hbm.at[pidx], buf_v, sem).wait()
            pltpu.sync_copy(
                buf_v, sv_hbm.at[hh, rr, pl.ds(cc * GCHUNK, GCHUNK)])


def _sc_route(qk_flat, v_flat, inv4):
    mesh = plsc.VectorSubcoreMesh(core_axis_name="c", subcore_axis_name="s")
    f = functools.partial(
        pl.kernel,
        mesh=mesh,
        compiler_params=pltpu.CompilerParams(needs_layout_passes=False,
                                             use_tc_tiling_on_sc=False),
        out_type=[
            jax.ShapeDtypeStruct((N_HEADS, N_HASHES, SEQ, D_HEAD), jnp.float32),
            jax.ShapeDtypeStruct((N_HEADS, N_HASHES, SEQ, D_HEAD), jnp.float32),
        ],
        scratch_types=[
            pltpu.VMEM((SEQ,), jnp.int32),
            pltpu.VMEM((SEQ,), jnp.int32),
            pltpu.VMEM((GCHUNK, D_HEAD), jnp.float32),
            pltpu.SemaphoreType.DMA,
        ],
    )(_sc_route_body)
    return f(qk_flat, v_flat, inv4)


def _bucket_body(sqk_ref, sv_ref, ob_ref):
    iota_r = lax.broadcasted_iota(jnp.int32, (ATT_GROUP, ATT_GROUP), 0)
    iota_cc = lax.broadcasted_iota(jnp.int32, (ATT_GROUP, ATT_GROUP), 1)
    same_chunk = (iota_r // BUCKET) == (iota_cc // BUCKET)
    for g in range(SEQ // ATT_GROUP):
        sq = sqk_ref[0, 0, g * ATT_GROUP:(g + 1) * ATT_GROUP].astype(
            jnp.bfloat16)
        sv = sv_ref[0, 0, g * ATT_GROUP:(g + 1) * ATT_GROUP].astype(
            jnp.bfloat16)
        s = lax.dot_general(sq, sq, (((1,), (1,)), ((), ())),
                            preferred_element_type=jnp.float32)
        s = jnp.where(same_chunk, s * SCALE, -1e9)
        m = jnp.max(s, axis=1, keepdims=True)
        e = jnp.exp(s - m)
        p = e / jnp.sum(e, axis=1, keepdims=True)
        ob_ref[0, 0, g * ATT_GROUP:(g + 1) * ATT_GROUP] = jnp.dot(
            p.astype(jnp.bfloat16), sv, preferred_element_type=jnp.float32)


def _stage3(sqk, sv):
    return pl.pallas_call(
        _bucket_body,
        grid=(N_HEADS, N_HASHES),
        in_specs=[
            pl.BlockSpec((1, 1, SEQ, D_HEAD), lambda h, r: (h, r, 0, 0)),
            pl.BlockSpec((1, 1, SEQ, D_HEAD), lambda h, r: (h, r, 0, 0)),
        ],
        out_specs=pl.BlockSpec((1, 1, SEQ, D_HEAD), lambda h, r: (h, r, 0, 0)),
        out_shape=jax.ShapeDtypeStruct((N_HEADS, N_HASHES, SEQ, D_HEAD),
                                       jnp.float32),
    )(sqk, sv)


def _sc_unsort_body(ob_hbm, scat_hbm, acc_hbm, idx_v, buf_v, sem):
    wid = lax.axis_index("s") * NC + lax.axis_index("c")
    for t in range(N_COMBOS // N_WORKERS):
        combo = wid * (N_COMBOS // N_WORKERS) + t
        hh = combo // N_HASHES
        rr = combo % N_HASHES
        pltpu.sync_copy(scat_hbm.at[hh, rr], idx_v)
        for cc in range(SEQ // GCHUNK):
            sidx = idx_v.at[pl.ds(cc * GCHUNK, GCHUNK)]
            pltpu.async_copy(ob_hbm.at[sidx], buf_v, sem).wait()
            pltpu.sync_copy(
                buf_v, acc_hbm.at[hh, rr, pl.ds(cc * GCHUNK, GCHUNK)])


def _sc_unsort(ob_flat, scat4):
    mesh = plsc.VectorSubcoreMesh(core_axis_name="c", subcore_axis_name="s")
    f = functools.partial(
        pl.kernel,
        mesh=mesh,
        compiler_params=pltpu.CompilerParams(needs_layout_passes=False,
                                             use_tc_tiling_on_sc=False),
        out_type=jax.ShapeDtypeStruct((N_HEADS, N_HASHES, SEQ, D_HEAD),
                                      jnp.float32),
        scratch_types=[
            pltpu.VMEM((SEQ,), jnp.int32),
            pltpu.VMEM((GCHUNK, D_HEAD), jnp.float32),
            pltpu.SemaphoreType.DMA,
        ],
    )(_sc_unsort_body)
    return f(ob_flat, scat4)


LCHUNK = 128
HALO = 8
HW = LCHUNK + 2 * HALO


def _mix_body(qk3_ref, v3_ref, acc_ref, wr_ref, mixed_ref, reg_ref):
    qk = qk3_ref[0]
    v = v3_ref[0]
    zp = jnp.zeros((HALO, D_HEAD), jnp.float32)
    qk_pad = jnp.concatenate([zp, qk, zp], axis=0).astype(jnp.bfloat16)
    v_pad = jnp.concatenate([zp, v, zp], axis=0).astype(jnp.bfloat16)
    qkb = qk.astype(jnp.bfloat16)
    di = (lax.broadcasted_iota(jnp.int32, (LCHUNK, HW), 1)
          - lax.broadcasted_iota(jnp.int32, (LCHUNK, HW), 0) - HALO)
    band = (di >= -RADIUS) & (di <= RADIUS)
    gcol_j = lax.broadcasted_iota(jnp.int32, (LCHUNK, HW), 1)
    locs = []
    for c in range(SEQ // LCHUNK):
        kh = qk_pad[c * LCHUNK:c * LCHUNK + HW]
        vh = v_pad[c * LCHUNK:c * LCHUNK + HW]
        s = lax.dot_general(qkb[c * LCHUNK:(c + 1) * LCHUNK], kh,
                            (((1,), (1,)), ((), ())),
                            preferred_element_type=jnp.float32)
        mask = band
        if c == 0 or c == SEQ // LCHUNK - 1:
            gcol = c * LCHUNK - HALO + gcol_j
            mask = mask & (gcol >= 0) & (gcol < SEQ)
        s = jnp.where(mask, s * SCALE, -1e9)
        m = jnp.max(s, axis=1, keepdims=True)
        e = jnp.exp(s - m)
        p = e / jnp.sum(e, axis=1, keepdims=True)
        locs.append(jnp.dot(p.astype(jnp.bfloat16), vh,
                            preferred_element_type=jnp.float32))
    local = jnp.concatenate(locs, axis=0)
    glin = jnp.dot(qk, wr_ref[0], preferred_element_type=jnp.float32)
    gm = jnp.max(glin, axis=1, keepdims=True)
    ge = jnp.exp(glin - gm)
    gate = ge / jnp.sum(ge, axis=1, keepdims=True)
    reg_ref[0] = jnp.sum(gate * jnp.log(gate + 1e-9), keepdims=True)
    lsh = (acc_ref[0, 0] + acc_ref[0, 1] + acc_ref[0, 2] + acc_ref[0, 3]) * 0.25
    mixed_ref[0] = (gate[:, 0:1] * local + gate[:, 1:2] * lsh).astype(
        jnp.bfloat16)


def _stage4(qk3, v3, acc, Wr):
    return pl.pallas_call(
        _mix_body,
        grid=(N_HEADS,),
        in_specs=[
            pl.BlockSpec((1, SEQ, D_HEAD), lambda h: (h, 0, 0)),
            pl.BlockSpec((1, SEQ, D_HEAD), lambda h: (h, 0, 0)),
            pl.BlockSpec((1, N_HASHES, SEQ, D_HEAD), lambda h: (h, 0, 0, 0)),
            pl.BlockSpec((1, D_HEAD, 2), lambda h: (h, 0, 0)),
        ],
        out_specs=[
            pl.BlockSpec((1, SEQ, D_HEAD), lambda h: (h, 0, 0)),
            pl.BlockSpec((1, 1, 1), lambda h: (h, 0, 0)),
        ],
        out_shape=[
            jax.ShapeDtypeStruct((N_HEADS, SEQ, D_HEAD), jnp.bfloat16),
            jax.ShapeDtypeStruct((N_HEADS, 1, 1), jnp.float32),
        ],
    )(qk3, v3, acc, Wr)


def _out_body(mixed_ref, x1_ref, x2_ref, wo_ref, w1_ref, b1_ref, w2_ref,
              b2_ref, g_ref, b_ref, gf_ref, gg_ref, y1_ref, y2_ref):
    mixed = jnp.concatenate([mixed_ref[hh] for hh in range(N_HEADS)], axis=1)
    attn = jnp.dot(mixed, wo_ref[...], preferred_element_type=jnp.float32)
    sig_f = 1.0 / (1.0 + jnp.exp(-gf_ref[...]))
    sig_g = 1.0 / (1.0 + jnp.exp(-gg_ref[...]))
    y1 = x1_ref[...] + sig_f * attn
    hn = _layernorm(y1, g_ref[...], b_ref[...])
    f = jnp.maximum(
        jnp.dot(hn.astype(jnp.bfloat16), w1_ref[...],
                preferred_element_type=jnp.float32) + b1_ref[...], 0.0)
    ffn = jnp.dot(f.astype(jnp.bfloat16), w2_ref[...],
                  preferred_element_type=jnp.float32) + b2_ref[...]
    y1_ref[...] = y1
    y2_ref[...] = x2_ref[...] + sig_g * ffn


def _stage5(mixed3, x1d, x2d, Wo, W1, b1, W2, b2, g, b, gf, gg):
    Wo = Wo.astype(jnp.bfloat16)
    W1 = W1.astype(jnp.bfloat16)
    W2 = W2.astype(jnp.bfloat16)
    vec = pl.BlockSpec((1, D_MODEL), lambda i: (0, 0))
    mat = pl.BlockSpec((D_MODEL, D_MODEL), lambda i: (0, 0))
    row = pl.BlockSpec((ROW_TILE, D_MODEL), lambda i: (i, 0))
    return pl.pallas_call(
        _out_body,
        grid=(N_ROW_TILES,),
        in_specs=[
            pl.BlockSpec((N_HEADS, ROW_TILE, D_HEAD), lambda i: (0, i, 0)),
            row, row, mat, mat, vec, mat, vec, vec, vec, vec, vec,
        ],
        out_specs=[row, row],
        out_shape=[
            jax.ShapeDtypeStruct((SEQ, D_MODEL), jnp.float32),
            jax.ShapeDtypeStruct((SEQ, D_MODEL), jnp.float32),
        ],
    )(mixed3, x1d, x2d, Wo, W1, b1, W2, b2, g, b, gf, gg)


def kernel(x1, x2, Wqk, Wv, Wo, Rh, Wr, ln_attn_g, ln_attn_b, ln_ffn_g,
           ln_ffn_b, W1, b1, W2, b2, gate_f, gate_g):
    x1d = x1[0]
    x2d = x2[0]
    qk3, v3 = _stage1(x2d, ln_attn_g[None], ln_attn_b[None], Wqk, Wv)
    RhT = Rh.transpose(0, 2, 1).reshape(N_HASHES * NB_HALF, D_HEAD)
    inv4, scat4 = _stage2(qk3, RhT)
    qk_flat = qk3.reshape(N_HEADS * SEQ, D_HEAD)
    v_flat = v3.reshape(N_HEADS * SEQ, D_HEAD)
    sqk, sv = _sc_route(qk_flat, v_flat, inv4)
    ob = _stage3(sqk, sv)
    ob_flat = ob.reshape(N_COMBOS * SEQ, D_HEAD)
    acc = _sc_unsort(ob_flat, scat4)
    mixed3, regs = _stage4(qk3, v3, acc, Wr)
    y1d, y2d = _stage5(mixed3, x1d, x2d, Wo, W1, b1[None], W2, b2[None],
                       ln_ffn_g[None], ln_ffn_b[None], gate_f[None],
                       gate_g[None])
    reg_loss = -jnp.sum(regs) / (N_HEADS * SEQ)
    return (y1d[None], y2d[None], reg_loss)
